# Initial kernel scaffold; baseline (speedup 1.0000x reference)
import jax
import jax.numpy as jnp
from jax.experimental import pallas as pl


def kernel(word_ids, table, gamma, beta):
    raise NotImplementedError("write your pallas kernel here")


# speedup vs baseline: 19.7540x; 19.7540x over previous
import functools

import jax
import jax.numpy as jnp
from jax import lax
from jax.experimental import pallas as pl
from jax.experimental.pallas import tpu as pltpu
from jax.experimental.pallas import tpu_sc as plsc

NUM_WORD = 100000
EMB_DIM = 64
BATCH = 16384
HIST = 50
EPS = 1e-5

NC = 2
NS = 16
NW = NC * NS
B_PER_W = BATCH // NW
G = 2
ROWS_PER_CHUNK = G * HIST
IDX_STRIDE = 104
N_CHUNKS = B_PER_W // G
TILE_IDX = N_CHUNKS * IDX_STRIDE
DMAS_PER_BUF = 2
BUF_ROWS = ROWS_PER_CHUNK * DMAS_PER_BUF
GB = G * DMAS_PER_BUF
N_FILLS = N_CHUNKS // DMAS_PER_BUF
NBUF = 4
N_OUTER = N_FILLS // NBUF
NVEC = EMB_DIM // 16
UNROLL = 5


def _rsqrt16(t):
    i = plsc.bitcast(t, jnp.int32)
    y = plsc.bitcast(jnp.int32(0x5F3759DF) - (i >> 1), jnp.float32)
    half = t * jnp.float32(0.5)
    for _ in range(3):
        y = y * (jnp.float32(1.5) - half * y * y)
    return y


def _sc_kernel():
    mesh = plsc.VectorSubcoreMesh(core_axis_name="c", subcore_axis_name="s")
    scratch = (
        [pltpu.VMEM((TILE_IDX,), jnp.int32)]
        + [pltpu.VMEM((BUF_ROWS, EMB_DIM), jnp.float32) for _ in range(NBUF)]
        + [
            pltpu.VMEM((B_PER_W, EMB_DIM), jnp.float32),
            pltpu.VMEM((EMB_DIM,), jnp.float32),
            pltpu.VMEM((EMB_DIM,), jnp.float32),
        ]
        + [pltpu.SemaphoreType.DMA for _ in range(NBUF)]
    )

    @functools.partial(
        pl.kernel,
        mesh=mesh,
        out_type=jax.ShapeDtypeStruct((BATCH, EMB_DIM), jnp.float32),
        scratch_types=scratch,
        compiler_params=pltpu.CompilerParams(
            use_tc_tiling_on_sc=False, needs_layout_passes=False
        ),
    )
    def body(ids_hbm, table_hbm, gamma_hbm, beta_hbm, out_hbm, idx_v, *rest):
        bufs = rest[:NBUF]
        out_v, gamma_v, beta_v = rest[NBUF : NBUF + 3]
        sems = rest[NBUF + 3 : NBUF + 3 + NBUF]

        wid = lax.axis_index("s") * NC + lax.axis_index("c")

        base = pl.multiple_of(wid * TILE_IDX, 8)
        pltpu.sync_copy(ids_hbm.at[pl.ds(base, TILE_IDX)], idx_v)
        pltpu.sync_copy(gamma_hbm, gamma_v)
        pltpu.sync_copy(beta_hbm, beta_v)
        gam = [gamma_v[pl.ds(16 * c, 16)] for c in range(NVEC)]
        bet = [beta_v[pl.ds(16 * c, 16)] for c in range(NVEC)]

        def start(fill, b):
            for h in range(DMAS_PER_BUF):
                off = pl.multiple_of(
                    (fill * DMAS_PER_BUF + h) * IDX_STRIDE, 8
                )
                pltpu.make_async_copy(
                    table_hbm.at[idx_v.at[pl.ds(off, ROWS_PER_CHUNK)]],
                    bufs[b].at[pl.ds(h * ROWS_PER_CHUNK, ROWS_PER_CHUNK)],
                    sems[b],
                ).start()

        def wait(b):
            for h in range(DMAS_PER_BUF):
                pltpu.make_async_copy(
                    table_hbm.at[idx_v.at[pl.ds(0, ROWS_PER_CHUNK)]],
                    bufs[b].at[pl.ds(h * ROWS_PER_CHUNK, ROWS_PER_CHUNK)],
                    sems[b],
                ).wait()

        for b in range(NBUF):
            start(b, b)

        scale = jnp.float32(1.0 / HIST)
        inv_d = jnp.float32(1.0 / EMB_DIM)

        def outer(g0, carry):
            for b in range(NBUF):
                fill = g0 * NBUF + b
                wait(b)
                for r in range(GB):
                    def abody(jj, acc):
                        acc = list(acc)
                        for u in range(UNROLL):
                            row = r * HIST + jj * UNROLL + u
                            for c in range(NVEC):
                                acc[c] = acc[c] + bufs[b][row, pl.ds(16 * c, 16)]
                        return tuple(acc)
                    acc = lax.fori_loop(
                        0, HIST // UNROLL, abody,
                        tuple(jnp.zeros((16,), jnp.float32) for _ in range(NVEC)),
                    )
                    x = [acc[c] * scale for c in range(NVEC)]
                    mu = jnp.sum(x[0] + x[1] + x[2] + x[3]) * inv_d
                    xc = [x[c] - mu for c in range(NVEC)]
                    sq = xc[0] * xc[0] + xc[1] * xc[1] + xc[2] * xc[2] + xc[3] * xc[3]
                    var = jnp.sum(sq) * inv_d
                    rs = _rsqrt16(jnp.full((16,), var + jnp.float32(EPS), jnp.float32))
                    orow = fill * GB + r
                    for c in range(NVEC):
                        out_v[orow, pl.ds(16 * c, 16)] = xc[c] * rs * gam[c] + bet[c]

                @pl.when(g0 < N_OUTER - 1)
                def _prefetch():
                    start(fill + NBUF, b)
            return carry

        lax.fori_loop(0, N_OUTER, outer, 0)

        pltpu.sync_copy(out_v, out_hbm.at[pl.ds(wid * B_PER_W, B_PER_W)])

    return body


def kernel(word_ids, table, gamma, beta):
    ids = word_ids.astype(jnp.int32).reshape(NW, N_CHUNKS, ROWS_PER_CHUNK)
    ids_pad = jnp.pad(ids, ((0, 0), (0, 0), (0, IDX_STRIDE - ROWS_PER_CHUNK)))
    return _sc_kernel()(ids_pad.reshape(-1), table, gamma, beta)


# speedup vs baseline: 21.8204x; 1.1046x over previous
import functools

import jax
import jax.numpy as jnp
from jax import lax
from jax.experimental import pallas as pl
from jax.experimental.pallas import tpu as pltpu
from jax.experimental.pallas import tpu_sc as plsc

NUM_WORD = 100000
EMB_DIM = 64
BATCH = 16384
HIST = 50
EPS = 1e-5

NC = 2
NS = 16
NW = NC * NS
B_PER_W = BATCH // NW
G = 2
ROWS_PER_CHUNK = G * HIST
N_CHUNKS = B_PER_W // G
DMAS_PER_BUF = 2
BUF_ROWS = ROWS_PER_CHUNK * DMAS_PER_BUF
GB = G * DMAS_PER_BUF
N_FILLS = N_CHUNKS // DMAS_PER_BUF
NBUF = 4
N_OUTER = N_FILLS // NBUF
NVEC = EMB_DIM // 16
UNROLL = 5


def _sc_pool():
    mesh = plsc.VectorSubcoreMesh(core_axis_name="c", subcore_axis_name="s")
    scratch = (
        [pltpu.VMEM((N_CHUNKS, ROWS_PER_CHUNK), jnp.int32)]
        + [pltpu.VMEM((BUF_ROWS, EMB_DIM), jnp.float32) for _ in range(NBUF)]
        + [pltpu.VMEM((B_PER_W, EMB_DIM), jnp.float32)]
        + [pltpu.SemaphoreType.DMA for _ in range(NBUF)]
    )

    @functools.partial(
        pl.kernel,
        mesh=mesh,
        out_type=jax.ShapeDtypeStruct((BATCH, EMB_DIM), jnp.float32),
        scratch_types=scratch,
        compiler_params=pltpu.CompilerParams(use_tc_tiling_on_sc=False),
    )
    def pool(ids_hbm, table_hbm, out_hbm, idx_v, *rest):
        bufs = rest[:NBUF]
        out_v = rest[NBUF]
        sems = rest[NBUF + 1 : NBUF + 1 + NBUF]

        wid = lax.axis_index("s") * NC + lax.axis_index("c")

        pltpu.sync_copy(ids_hbm.at[wid], idx_v)

        def start(fill, b):
            for h in range(DMAS_PER_BUF):
                pltpu.make_async_copy(
                    table_hbm.at[idx_v.at[fill * DMAS_PER_BUF + h]],
                    bufs[b].at[pl.ds(h * ROWS_PER_CHUNK, ROWS_PER_CHUNK)],
                    sems[b],
                ).start()

        def wait(b):
            for h in range(DMAS_PER_BUF):
                pltpu.make_async_copy(
                    table_hbm.at[idx_v.at[0]],
                    bufs[b].at[pl.ds(h * ROWS_PER_CHUNK, ROWS_PER_CHUNK)],
                    sems[b],
                ).wait()

        for b in range(NBUF):
            start(b, b)

        scale = jnp.float32(1.0 / HIST)

        def outer(g0, carry):
            for b in range(NBUF):
                fill = g0 * NBUF + b
                wait(b)
                for r in range(GB):
                    def body(jj, acc):
                        acc = list(acc)
                        for u in range(UNROLL):
                            row = r * HIST + jj * UNROLL + u
                            for c in range(NVEC):
                                acc[c] = acc[c] + bufs[b][row, pl.ds(16 * c, 16)]
                        return tuple(acc)
                    acc = lax.fori_loop(
                        0, HIST // UNROLL, body,
                        tuple(jnp.zeros((16,), jnp.float32) for _ in range(NVEC)),
                    )
                    orow = fill * GB + r
                    for c in range(NVEC):
                        out_v[orow, pl.ds(16 * c, 16)] = acc[c] * scale

                @pl.when(g0 < N_OUTER - 1)
                def _prefetch():
                    start(fill + NBUF, b)
            return carry

        lax.fori_loop(0, N_OUTER, outer, 0)

        pltpu.sync_copy(out_v, out_hbm.at[pl.ds(wid * B_PER_W, B_PER_W)])

    return pool


_LN_BLK = 1024


def _ln_body(x_ref, g_ref, b_ref, o_ref):
    x = x_ref[...]
    mu = jnp.mean(x, axis=-1, keepdims=True)
    d = x - mu
    var = jnp.mean(d * d, axis=-1, keepdims=True)
    o_ref[...] = d * lax.rsqrt(var + EPS) * g_ref[...] + b_ref[...]


def _layernorm(x, gamma, beta):
    return pl.pallas_call(
        _ln_body,
        grid=(BATCH // _LN_BLK,),
        in_specs=[
            pl.BlockSpec((_LN_BLK, EMB_DIM), lambda i: (i, 0)),
            pl.BlockSpec((EMB_DIM,), lambda i: (0,)),
            pl.BlockSpec((EMB_DIM,), lambda i: (0,)),
        ],
        out_specs=pl.BlockSpec((_LN_BLK, EMB_DIM), lambda i: (i, 0)),
        out_shape=jax.ShapeDtypeStruct((BATCH, EMB_DIM), jnp.float32),
    )(x, gamma, beta)


def kernel(word_ids, table, gamma, beta):
    ids3 = word_ids.reshape(NW, N_CHUNKS, ROWS_PER_CHUNK).astype(jnp.int32)
    pooled = _sc_pool()(ids3, table)
    return _layernorm(pooled, gamma, beta)
